# Initial kernel scaffold; baseline (speedup 1.0000x reference)
import jax
import jax.numpy as jnp
from jax.experimental import pallas as pl


def kernel(node_features, edge_features, senders, receivers, We1, be1, We2, be2, ge, bge, Wn1, bn1, Wn2, bn2, gn, bgn):
    raise NotImplementedError("write your pallas kernel here")


# speedup vs baseline: 3.6034x; 3.6034x over previous
import functools

import jax
import jax.numpy as jnp
from jax import lax
from jax.experimental import pallas as pl
from jax.experimental.pallas import tpu as pltpu
from jax.experimental.pallas import tpu_sc as plsc

_N = 10000
_E = 320000
_D = 128

_NC = 2
_NS = 16
_NW = _NC * _NS
_EPW = _E // _NW
_CH = 80
_NCH = _EPW // _CH
_RPT = 624
_RTAIL = _N - _RPT * _NS

_EBLK = 1600


def _proj_body(nf_ref, ws_ref, wr_ref, ps_ref, pr_ref):
    nf = nf_ref[...]
    ps_ref[...] = jnp.dot(nf, ws_ref[...], preferred_element_type=jnp.float32)
    pr_ref[...] = jnp.dot(nf, wr_ref[...], preferred_element_type=jnp.float32)


def _edge_body(g1_ref, g2_ref, ef_ref, w1_ref, b1_ref, w2_ref, b2_ref,
               gg_ref, bg_ref, ne_ref, eo_ref):
    ef = ef_ref[...]
    x = (g1_ref[...] + g2_ref[...] + b1_ref[...]
         + jnp.dot(ef, w1_ref[...], preferred_element_type=jnp.float32))
    h = jnp.maximum(x, 0.0)
    h2 = jnp.dot(h, w2_ref[...], preferred_element_type=jnp.float32) + b2_ref[...]
    mu = jnp.mean(h2, axis=-1, keepdims=True)
    c = h2 - mu
    var = jnp.mean(c * c, axis=-1, keepdims=True)
    ln = gg_ref[...] * c * lax.rsqrt(var + 1e-5) + bg_ref[...]
    ne_ref[...] = ln
    eo_ref[...] = ln + ef


def _node_body(nf_ref, a0_ref, a1_ref, w1a_ref, w1b_ref, b1_ref, w2_ref,
               b2_ref, gg_ref, bg_ref, out_ref):
    nf = nf_ref[...]
    agg = a0_ref[...] + a1_ref[...]
    x = (jnp.dot(nf, w1a_ref[...], preferred_element_type=jnp.float32)
         + jnp.dot(agg, w1b_ref[...], preferred_element_type=jnp.float32)
         + b1_ref[...])
    h = jnp.maximum(x, 0.0)
    h2 = jnp.dot(h, w2_ref[...], preferred_element_type=jnp.float32) + b2_ref[...]
    mu = jnp.mean(h2, axis=-1, keepdims=True)
    c = h2 - mu
    var = jnp.mean(c * c, axis=-1, keepdims=True)
    ln = gg_ref[...] * c * lax.rsqrt(var + 1e-5) + bg_ref[...]
    out_ref[...] = ln + nf


_proj = pl.pallas_call(
    _proj_body,
    out_shape=(jax.ShapeDtypeStruct((_N, _D), jnp.float32),
               jax.ShapeDtypeStruct((_N, _D), jnp.float32)),
)

_full = pl.BlockSpec((_D, _D), lambda i: (0, 0))
_vec = pl.BlockSpec((1, _D), lambda i: (0, 0))
_eblk = pl.BlockSpec((_EBLK, _D), lambda i: (i, 0))

_edge_mlp = pl.pallas_call(
    _edge_body,
    grid=(_E // _EBLK,),
    in_specs=[_eblk, _eblk, _eblk, _full, _vec, _full, _vec, _vec, _vec],
    out_specs=(_eblk, _eblk),
    out_shape=(jax.ShapeDtypeStruct((_E, _D), jnp.float32),
               jax.ShapeDtypeStruct((_E, _D), jnp.float32)),
)

_node_mlp = pl.pallas_call(
    _node_body,
    out_shape=jax.ShapeDtypeStruct((_N, _D), jnp.float32),
)


def _sc_gather_body(ps_hbm, pr_hbm, snd_hbm, rcv_hbm, g1_hbm, g2_hbm,
                    idx_s, idx_r, rows_s, rows_r, sem_s, sem_r):
    cid = lax.axis_index("c")
    sid = lax.axis_index("s")
    wid = sid * _NC + cid
    pltpu.sync_copy(snd_hbm.at[wid], idx_s)
    pltpu.sync_copy(rcv_hbm.at[wid], idx_r)
    base = wid * _EPW

    def body(i, carry):
        off = pl.multiple_of(base + i * _CH, 8)
        cp1 = pltpu.async_copy(ps_hbm.at[idx_s.at[i]], rows_s, sem_s)
        cp2 = pltpu.async_copy(pr_hbm.at[idx_r.at[i]], rows_r, sem_r)
        cp1.wait()
        pltpu.sync_copy(rows_s, g1_hbm.at[pl.ds(off, _CH)])
        cp2.wait()
        pltpu.sync_copy(rows_r, g2_hbm.at[pl.ds(off, _CH)])
        return carry

    lax.fori_loop(0, _NCH, body, 0)


def _sc_scatter_body(ne_hbm, rcv_hbm, zeros_hbm, out_hbm, idx_v, rows_v, agg_sh):
    cid = lax.axis_index("c")
    sid = lax.axis_index("s")
    wid = sid * _NC + cid
    r0 = pl.multiple_of(sid * _RPT, 8)
    pltpu.sync_copy(zeros_hbm.at[pl.ds(r0, _RPT)], agg_sh.at[pl.ds(r0, _RPT)])

    @pl.when(sid == 0)
    def _():
        pltpu.sync_copy(zeros_hbm.at[pl.ds(_RPT * _NS, _RTAIL)],
                        agg_sh.at[pl.ds(_RPT * _NS, _RTAIL)])

    pltpu.sync_copy(rcv_hbm.at[wid], idx_v)
    plsc.subcore_barrier()
    base = wid * _EPW

    def body(i, carry):
        off = pl.multiple_of(base + i * _CH, 8)
        pltpu.sync_copy(ne_hbm.at[pl.ds(off, _CH)], rows_v)
        pltpu.sync_copy(rows_v, agg_sh.at[idx_v.at[i]], add=True)
        return carry

    lax.fori_loop(0, _NCH, body, 0)
    plsc.subcore_barrier()
    out0 = pl.multiple_of(cid * _N + r0, 8)
    pltpu.sync_copy(agg_sh.at[pl.ds(r0, _RPT)], out_hbm.at[pl.ds(out0, _RPT)])

    @pl.when(sid == 0)
    def _():
        tail0 = pl.multiple_of(cid * _N + _RPT * _NS, 8)
        pltpu.sync_copy(agg_sh.at[pl.ds(_RPT * _NS, _RTAIL)],
                        out_hbm.at[pl.ds(tail0, _RTAIL)])


@functools.lru_cache(maxsize=None)
def _sc_kernels():
    mesh = plsc.VectorSubcoreMesh(
        core_axis_name="c", subcore_axis_name="s",
        num_cores=_NC, num_subcores=_NS)
    gather = pl.kernel(
        _sc_gather_body,
        out_type=(jax.ShapeDtypeStruct((_E, _D), jnp.float32),
                  jax.ShapeDtypeStruct((_E, _D), jnp.float32)),
        mesh=mesh,
        scratch_types=[
            pltpu.VMEM((_NCH, _CH), jnp.int32),
            pltpu.VMEM((_NCH, _CH), jnp.int32),
            pltpu.VMEM((_CH, _D), jnp.float32),
            pltpu.VMEM((_CH, _D), jnp.float32),
            pltpu.SemaphoreType.DMA,
            pltpu.SemaphoreType.DMA,
        ],
    )
    scatter = pl.kernel(
        _sc_scatter_body,
        out_type=jax.ShapeDtypeStruct((_NC * _N, _D), jnp.float32),
        mesh=mesh,
        scratch_types=[
            pltpu.VMEM((_NCH, _CH), jnp.int32),
            pltpu.VMEM((_CH, _D), jnp.float32),
            pltpu.VMEM_SHARED((_N, _D), jnp.float32),
        ],
    )
    return gather, scatter


def kernel(node_features, edge_features, senders, receivers, We1, be1, We2,
           be2, ge, bge, Wn1, bn1, Wn2, bn2, gn, bgn):
    sc_gather, sc_scatter = _sc_kernels()
    ps, pr = _proj(node_features, We1[:_D], We1[_D:2 * _D])
    snd3 = senders.reshape(_NW, _NCH, _CH)
    rcv3 = receivers.reshape(_NW, _NCH, _CH)
    g1, g2 = sc_gather(ps, pr, snd3, rcv3)
    new_edge, edge_out = _edge_mlp(
        g1, g2, edge_features, We1[2 * _D:], be1.reshape(1, _D), We2,
        be2.reshape(1, _D), ge.reshape(1, _D), bge.reshape(1, _D))
    zeros = jnp.zeros((_N, _D), jnp.float32)
    agg2 = sc_scatter(new_edge, rcv3, zeros)
    new_node = _node_mlp(
        node_features, agg2[:_N], agg2[_N:], Wn1[:_D], Wn1[_D:],
        bn1.reshape(1, _D), Wn2, bn2.reshape(1, _D), gn.reshape(1, _D),
        bgn.reshape(1, _D))
    return new_node, edge_out


# speedup vs baseline: 5.1076x; 1.4175x over previous
import functools

import jax
import jax.numpy as jnp
from jax import lax
from jax.experimental import pallas as pl
from jax.experimental.pallas import tpu as pltpu
from jax.experimental.pallas import tpu_sc as plsc

_N = 10000
_E = 320000
_D = 128

_NC = 2
_NS = 16
_NW = _NC * _NS
_EPW = _E // _NW
_CH = 80
_NCH = _EPW // _CH
_RPT = 624
_RTAIL = _N - _RPT * _NS

_EBLK = 1600


def _proj_body(nf_ref, ws_ref, wr_ref, ps_ref, pr_ref):
    nf = nf_ref[...]
    ps_ref[...] = jnp.dot(nf, ws_ref[...], preferred_element_type=jnp.float32)
    pr_ref[...] = jnp.dot(nf, wr_ref[...], preferred_element_type=jnp.float32)


def _edge_body(g_ref, ef_ref, w1_ref, b1_ref, w2_ref, b2_ref,
               gg_ref, bg_ref, ne_ref, eo_ref):
    ef = ef_ref[...]
    x = (g_ref[...] + b1_ref[...]
         + jnp.dot(ef, w1_ref[...], preferred_element_type=jnp.float32))
    h = jnp.maximum(x, 0.0)
    h2 = jnp.dot(h, w2_ref[...], preferred_element_type=jnp.float32) + b2_ref[...]
    mu = jnp.mean(h2, axis=-1, keepdims=True)
    c = h2 - mu
    var = jnp.mean(c * c, axis=-1, keepdims=True)
    ln = gg_ref[...] * c * lax.rsqrt(var + 1e-5) + bg_ref[...]
    ne_ref[...] = ln
    eo_ref[...] = ln + ef


def _node_body(nf_ref, a0_ref, a1_ref, w1a_ref, w1b_ref, b1_ref, w2_ref,
               b2_ref, gg_ref, bg_ref, out_ref):
    nf = nf_ref[...]
    agg = a0_ref[...] + a1_ref[...]
    x = (jnp.dot(nf, w1a_ref[...], preferred_element_type=jnp.float32)
         + jnp.dot(agg, w1b_ref[...], preferred_element_type=jnp.float32)
         + b1_ref[...])
    h = jnp.maximum(x, 0.0)
    h2 = jnp.dot(h, w2_ref[...], preferred_element_type=jnp.float32) + b2_ref[...]
    mu = jnp.mean(h2, axis=-1, keepdims=True)
    c = h2 - mu
    var = jnp.mean(c * c, axis=-1, keepdims=True)
    ln = gg_ref[...] * c * lax.rsqrt(var + 1e-5) + bg_ref[...]
    out_ref[...] = ln + nf


_proj = pl.pallas_call(
    _proj_body,
    out_shape=(jax.ShapeDtypeStruct((_N, _D), jnp.float32),
               jax.ShapeDtypeStruct((_N, _D), jnp.float32)),
)

_full = pl.BlockSpec((_D, _D), lambda i: (0, 0))
_vec = pl.BlockSpec((1, _D), lambda i: (0, 0))
_eblk = pl.BlockSpec((_EBLK, _D), lambda i: (i, 0))

_edge_mlp = pl.pallas_call(
    _edge_body,
    grid=(_E // _EBLK,),
    in_specs=[_eblk, _eblk, _full, _vec, _full, _vec, _vec, _vec],
    out_specs=(_eblk, _eblk),
    out_shape=(jax.ShapeDtypeStruct((_E, _D), jnp.float32),
               jax.ShapeDtypeStruct((_E, _D), jnp.float32)),
)

_node_mlp = pl.pallas_call(
    _node_body,
    out_shape=jax.ShapeDtypeStruct((_N, _D), jnp.float32),
)


def _sc_gather_body(ps_hbm, pr_hbm, snd_hbm, rcv_hbm, g_hbm,
                    idx_s, idx_r, rows_s0, rows_r0, rows_s1, rows_r1,
                    rows_s2, rows_r2, out0, out1, out2,
                    sem_s0, sem_r0, sem_s1, sem_r1, sem_s2, sem_r2,
                    sem_o0, sem_o1, sem_o2):
    cid = lax.axis_index("c")
    sid = lax.axis_index("s")
    wid = sid * _NC + cid
    pltpu.sync_copy(snd_hbm.at[wid], idx_s)
    pltpu.sync_copy(rcv_hbm.at[wid], idx_r)
    base = wid * _EPW

    def issue(i, rows_s, rows_r, sem_s, sem_r):
        pltpu.async_copy(ps_hbm.at[idx_s.at[i]], rows_s, sem_s)
        pltpu.async_copy(pr_hbm.at[idx_r.at[i]], rows_r, sem_r)

    def step(i, rows_s, rows_r, out, sem_s, sem_r, sem_o):
        pltpu.make_async_copy(ps_hbm.at[idx_s.at[i]], rows_s, sem_s).wait()
        pltpu.make_async_copy(pr_hbm.at[idx_r.at[i]], rows_r, sem_r).wait()
        off = pl.multiple_of(base + i * _CH, 8)

        @pl.when(i >= 3)
        def _():
            pltpu.make_async_copy(out, g_hbm.at[pl.ds(off, _CH)], sem_o).wait()

        @plsc.parallel_loop(0, _CH, unroll=4)
        def _(j):
            for c2 in range(_D // 16):
                sl = pl.ds(c2 * 16, 16)
                out[j, sl] = rows_s[j, sl] + rows_r[j, sl]

        @pl.when(i + 3 < _NCH)
        def _():
            issue(i + 3, rows_s, rows_r, sem_s, sem_r)

        pltpu.async_copy(out, g_hbm.at[pl.ds(off, _CH)], sem_o)

    slot0 = (rows_s0, rows_r0, out0, sem_s0, sem_r0, sem_o0)
    slot1 = (rows_s1, rows_r1, out1, sem_s1, sem_r1, sem_o1)
    slot2 = (rows_s2, rows_r2, out2, sem_s2, sem_r2, sem_o2)
    issue(0, rows_s0, rows_r0, sem_s0, sem_r0)
    issue(1, rows_s1, rows_r1, sem_s1, sem_r1)
    issue(2, rows_s2, rows_r2, sem_s2, sem_r2)

    def body(k, carry):
        i0 = k * 3
        step(i0, *slot0)
        step(i0 + 1, *slot1)
        step(i0 + 2, *slot2)
        return carry

    lax.fori_loop(0, _NCH // 3, body, 0)
    step(_NCH - 2, *slot0)
    step(_NCH - 1, *slot1)
    base8 = pl.multiple_of(base, 8)
    pltpu.make_async_copy(out0, g_hbm.at[pl.ds(base8, _CH)], sem_o0).wait()
    pltpu.make_async_copy(out1, g_hbm.at[pl.ds(base8, _CH)], sem_o1).wait()
    pltpu.make_async_copy(out2, g_hbm.at[pl.ds(base8, _CH)], sem_o2).wait()


def _sc_scatter_body(ne_hbm, rcv_hbm, zeros_hbm, out_hbm, idx_v,
                     rows0, rows1, rows2, agg_sh, sem0, sem1, sem2):
    cid = lax.axis_index("c")
    sid = lax.axis_index("s")
    wid = sid * _NC + cid
    r0 = pl.multiple_of(sid * _RPT, 8)
    pltpu.sync_copy(zeros_hbm.at[pl.ds(r0, _RPT)], agg_sh.at[pl.ds(r0, _RPT)])

    @pl.when(sid == 0)
    def _():
        pltpu.sync_copy(zeros_hbm.at[pl.ds(_RPT * _NS, _RTAIL)],
                        agg_sh.at[pl.ds(_RPT * _NS, _RTAIL)])

    pltpu.sync_copy(rcv_hbm.at[wid], idx_v)
    plsc.subcore_barrier()
    base = wid * _EPW

    def issue(i, rows, sem):
        off = pl.multiple_of(base + i * _CH, 8)
        pltpu.async_copy(ne_hbm.at[pl.ds(off, _CH)], rows, sem)

    def step(i, rows, sem):
        off = pl.multiple_of(base + i * _CH, 8)
        pltpu.make_async_copy(ne_hbm.at[pl.ds(off, _CH)], rows, sem).wait()
        pltpu.sync_copy(rows, agg_sh.at[idx_v.at[i]], add=True)

        @pl.when(i + 3 < _NCH)
        def _():
            issue(i + 3, rows, sem)

    issue(0, rows0, sem0)
    issue(1, rows1, sem1)
    issue(2, rows2, sem2)

    def body(k, carry):
        i0 = k * 3
        step(i0, rows0, sem0)
        step(i0 + 1, rows1, sem1)
        step(i0 + 2, rows2, sem2)
        return carry

    lax.fori_loop(0, _NCH // 3, body, 0)
    step(_NCH - 2, rows0, sem0)
    step(_NCH - 1, rows1, sem1)
    plsc.subcore_barrier()
    out0 = pl.multiple_of(cid * _N + r0, 8)
    pltpu.sync_copy(agg_sh.at[pl.ds(r0, _RPT)], out_hbm.at[pl.ds(out0, _RPT)])

    @pl.when(sid == 0)
    def _():
        tail0 = pl.multiple_of(cid * _N + _RPT * _NS, 8)
        pltpu.sync_copy(agg_sh.at[pl.ds(_RPT * _NS, _RTAIL)],
                        out_hbm.at[pl.ds(tail0, _RTAIL)])


@functools.lru_cache(maxsize=None)
def _sc_kernels():
    mesh = plsc.VectorSubcoreMesh(
        core_axis_name="c", subcore_axis_name="s",
        num_cores=_NC, num_subcores=_NS)
    gather = pl.kernel(
        _sc_gather_body,
        out_type=jax.ShapeDtypeStruct((_E, _D), jnp.float32),
        mesh=mesh,
        scratch_types=(
            [pltpu.VMEM((_NCH, _CH), jnp.int32)] * 2
            + [pltpu.VMEM((_CH, _D), jnp.float32)] * 9
            + [pltpu.SemaphoreType.DMA] * 9
        ),
    )
    scatter = pl.kernel(
        _sc_scatter_body,
        out_type=jax.ShapeDtypeStruct((_NC * _N, _D), jnp.float32),
        mesh=mesh,
        scratch_types=(
            [pltpu.VMEM((_NCH, _CH), jnp.int32)]
            + [pltpu.VMEM((_CH, _D), jnp.float32)] * 3
            + [pltpu.VMEM_SHARED((_N, _D), jnp.float32)]
            + [pltpu.SemaphoreType.DMA] * 3
        ),
    )
    return gather, scatter


def kernel(node_features, edge_features, senders, receivers, We1, be1, We2,
           be2, ge, bge, Wn1, bn1, Wn2, bn2, gn, bgn):
    sc_gather, sc_scatter = _sc_kernels()
    ps, pr = _proj(node_features, We1[:_D], We1[_D:2 * _D])
    snd3 = senders.reshape(_NW, _NCH, _CH)
    rcv3 = receivers.reshape(_NW, _NCH, _CH)
    g = sc_gather(ps, pr, snd3, rcv3)
    new_edge, edge_out = _edge_mlp(
        g, edge_features, We1[2 * _D:], be1.reshape(1, _D), We2,
        be2.reshape(1, _D), ge.reshape(1, _D), bge.reshape(1, _D))
    zeros = jnp.zeros((_N, _D), jnp.float32)
    agg2 = sc_scatter(new_edge, rcv3, zeros)
    new_node = _node_mlp(
        node_features, agg2[:_N], agg2[_N:], Wn1[:_D], Wn1[_D:],
        bn1.reshape(1, _D), Wn2, bn2.reshape(1, _D), gn.reshape(1, _D),
        bgn.reshape(1, _D))
    return new_node, edge_out
